# baseline (speedup 1.0000x reference)
import functools

import jax
import jax.numpy as jnp
from jax import lax
from jax.experimental import pallas as pl
from jax.experimental.pallas import tpu as pltpu
from jax.experimental.pallas import tpu_sc as plsc

L = 16
NC = 2
NS = 16
NW = NC * NS
SB = 1024


def _sc_lookup(n_tokens: int, vocab: int):
    per_w = n_tokens // NW
    n_j = per_w // SB
    mesh = plsc.VectorSubcoreMesh(core_axis_name="c", subcore_axis_name="s")

    @functools.partial(
        pl.kernel,
        out_type=jax.ShapeDtypeStruct((n_tokens,), jnp.int32),
        mesh=mesh,
        scratch_types=[
            pltpu.VMEM((vocab,), jnp.int32),
            pltpu.VMEM((per_w,), jnp.int32),
            pltpu.VMEM((per_w,), jnp.int32),
        ],
        compiler_params=pltpu.CompilerParams(needs_layout_passes=False),
    )
    def sc_kernel(ids_hbm, table_hbm, next_hbm, table_v, ids_v, next_v):
        wid = lax.axis_index("s") * NC + lax.axis_index("c")
        base = wid * per_w
        pltpu.sync_copy(table_hbm, table_v)
        pltpu.sync_copy(ids_hbm.at[pl.ds(base, per_w)], ids_v)

        iota = lax.iota(jnp.int32, L)
        basis = (iota % 8) * 128 + iota // 8

        @pl.loop(0, n_j)
        def _(j):
            @pl.loop(0, SB // L, unroll=8)
            def _(gi):
                g16 = j * (SB // L) + gi
                ids16 = ids_v[pl.ds(g16 * L, L)]
                next16 = plsc.load_gather(table_v, [ids16])
                plsc.store_scatter(next_v, [basis + (j * SB + gi * 2)], next16)

        pltpu.sync_copy(next_v, next_hbm.at[pl.ds(base, per_w)])

    return sc_kernel


def _tc_onehot(batch: int, seq: int, vocab: int):
    def body(fill_ref, nx_ref, out_ref):
        f0 = fill_ref[0]
        f1 = fill_ref[1]
        vio = lax.broadcasted_iota(jnp.int32, (8, vocab), 1)
        for c in range(128):
            nxc = nx_ref[0, 0, :, pl.ds(c, 1)]
            m = jnp.broadcast_to(nxc, (8, vocab)) == vio
            out_ref[0, pl.ds(c * 8, 8), :] = jnp.where(m, f1, f0)

    return pl.pallas_call(
        body,
        grid=(batch, seq // SB),
        in_specs=[
            pl.BlockSpec(memory_space=pltpu.SMEM),
            pl.BlockSpec((1, 1, 8, 128), lambda b, j: (b, j, 0, 0)),
        ],
        out_specs=pl.BlockSpec((1, SB, vocab), lambda b, j: (b, j, 0)),
        out_shape=jax.ShapeDtypeStruct((batch, seq, vocab), jnp.float32),
    )


def kernel(input_ids, transition_table, fill_values):
    batch, seq = input_ids.shape
    vocab = transition_table.shape[0]
    n = batch * seq
    ids_flat = input_ids.reshape(n)
    next_flat = _sc_lookup(n, vocab)(ids_flat, transition_table)
    nx4 = next_flat.reshape(batch, seq // SB, 8, 128)
    return _tc_onehot(batch, seq, vocab)(fill_values, nx4)


# speedup vs baseline: 1.8210x; 1.8210x over previous
import functools

import jax
import jax.numpy as jnp
from jax import lax
from jax.experimental import pallas as pl
from jax.experimental.pallas import tpu as pltpu
from jax.experimental.pallas import tpu_sc as plsc

L = 16
NC = 2
NS = 16
NW = NC * NS
SB = 1024


def _sc_lookup(n_tokens: int, vocab: int):
    per_w = n_tokens // NW
    mesh = plsc.VectorSubcoreMesh(core_axis_name="c", subcore_axis_name="s")

    @functools.partial(
        pl.kernel,
        out_type=jax.ShapeDtypeStruct((n_tokens,), jnp.int32),
        mesh=mesh,
        scratch_types=[
            pltpu.VMEM((vocab,), jnp.int32),
            pltpu.VMEM((per_w,), jnp.int32),
            pltpu.VMEM((per_w,), jnp.int32),
        ],
        compiler_params=pltpu.CompilerParams(needs_layout_passes=False),
    )
    def sc_kernel(ids_hbm, table_hbm, next_hbm, table_v, ids_v, next_v):
        wid = lax.axis_index("s") * NC + lax.axis_index("c")
        base = wid * per_w
        pltpu.sync_copy(table_hbm, table_v)
        pltpu.sync_copy(ids_hbm.at[pl.ds(base, per_w)], ids_v)

        @pl.loop(0, per_w // L, unroll=8)
        def _(j):
            ids16 = ids_v[pl.ds(j * L, L)]
            next_v[pl.ds(j * L, L)] = plsc.load_gather(table_v, [ids16])

        pltpu.sync_copy(next_v, next_hbm.at[pl.ds(base, per_w)])

    return sc_kernel


def _tc_onehot(batch: int, seq: int, vocab: int):
    def body(fill_ref, nx_ref, out_ref):
        f0 = fill_ref[0]
        f1 = fill_ref[1]
        vio = lax.broadcasted_iota(jnp.int32, (vocab, 128), 0)
        for g in range(SB // 128):
            row = nx_ref[0, pl.ds(g, 1), :]
            bc = jnp.broadcast_to(row, (vocab, 128))
            out_ref[0, :, pl.ds(g * 128, 128)] = jnp.where(bc == vio, f1, f0)

    return pl.pallas_call(
        body,
        grid=(batch, seq // SB),
        in_specs=[
            pl.BlockSpec(memory_space=pltpu.SMEM),
            pl.BlockSpec((1, SB // 128, 128), lambda b, j: (b, j, 0)),
        ],
        out_specs=pl.BlockSpec((1, vocab, SB), lambda b, j: (b, 0, j)),
        out_shape=jax.ShapeDtypeStruct((batch, vocab, seq), jnp.float32),
    )


def kernel(input_ids, transition_table, fill_values):
    batch, seq = input_ids.shape
    vocab = transition_table.shape[0]
    n = batch * seq
    ids_flat = input_ids.reshape(n)
    next_flat = _sc_lookup(n, vocab)(ids_flat, transition_table)
    nx3 = next_flat.reshape(batch, seq // 128, 128)
    out_t = _tc_onehot(batch, seq, vocab)(fill_values, nx3)
    return jnp.transpose(out_t, (0, 2, 1))


# speedup vs baseline: 5.4683x; 3.0029x over previous
import functools

import jax
import jax.numpy as jnp
from jax import lax
from jax.experimental import pallas as pl
from jax.experimental.pallas import tpu as pltpu
from jax.experimental.pallas import tpu_sc as plsc

L = 16
NC = 2
NS = 16
NW = NC * NS
SB = 8192


def _sc_lookup(n_tokens: int, vocab: int):
    per_w = n_tokens // NW
    mesh = plsc.VectorSubcoreMesh(core_axis_name="c", subcore_axis_name="s")

    @functools.partial(
        pl.kernel,
        out_type=jax.ShapeDtypeStruct((n_tokens,), jnp.int32),
        mesh=mesh,
        scratch_types=[
            pltpu.VMEM((vocab,), jnp.int32),
            pltpu.VMEM((per_w,), jnp.int32),
            pltpu.VMEM((per_w,), jnp.int32),
        ],
        compiler_params=pltpu.CompilerParams(needs_layout_passes=False),
    )
    def sc_kernel(ids_hbm, table_hbm, next_hbm, table_v, ids_v, next_v):
        wid = lax.axis_index("s") * NC + lax.axis_index("c")
        base = wid * per_w
        pltpu.sync_copy(table_hbm, table_v)
        pltpu.sync_copy(ids_hbm.at[pl.ds(base, per_w)], ids_v)

        @pl.loop(0, per_w // L, unroll=8)
        def _(j):
            ids16 = ids_v[pl.ds(j * L, L)]
            next_v[pl.ds(j * L, L)] = plsc.load_gather(table_v, [ids16])

        pltpu.sync_copy(next_v, next_hbm.at[pl.ds(base, per_w)])

    return sc_kernel


def _tc_onehot(batch: int, seq: int, vocab: int):
    def body(fill_ref, nx_ref, out_ref):
        f0 = fill_ref[0]
        f1 = fill_ref[1]
        vio = lax.broadcasted_iota(jnp.int32, (vocab, 128), 0)
        for g in range(SB // 128):
            row = nx_ref[0, pl.ds(g, 1), :]
            bc = jnp.broadcast_to(row, (vocab, 128))
            out_ref[0, :, pl.ds(g * 128, 128)] = jnp.where(bc == vio, f1, f0)

    return pl.pallas_call(
        body,
        grid=(batch,),
        in_specs=[
            pl.BlockSpec(memory_space=pltpu.SMEM),
            pl.BlockSpec((1, SB // 128, 128), lambda b: (b, 0, 0)),
        ],
        out_specs=pl.BlockSpec((1, vocab, SB), lambda b: (b, 0, 0)),
        out_shape=jax.ShapeDtypeStruct((batch, vocab, seq), jnp.float32),
    )


def kernel(input_ids, transition_table, fill_values):
    batch, seq = input_ids.shape
    vocab = transition_table.shape[0]
    n = batch * seq
    ids_flat = input_ids.reshape(n)
    next_flat = _sc_lookup(n, vocab)(ids_flat, transition_table)
    nx3 = next_flat.reshape(batch, seq // 128, 128)
    out_t = _tc_onehot(batch, seq, vocab)(fill_values, nx3)
    return jnp.transpose(out_t, (0, 2, 1))


# speedup vs baseline: 5.5225x; 1.0099x over previous
import functools

import jax
import jax.numpy as jnp
from jax import lax
from jax.experimental import pallas as pl
from jax.experimental.pallas import tpu as pltpu
from jax.experimental.pallas import tpu_sc as plsc

L = 16
NC = 2
NS = 16
NW = NC * NS
SB = 8192


def _sc_lookup(n_tokens: int, vocab: int):
    per_w = n_tokens // NW
    CH = per_w // 4
    mesh = plsc.VectorSubcoreMesh(core_axis_name="c", subcore_axis_name="s")

    @functools.partial(
        pl.kernel,
        out_type=jax.ShapeDtypeStruct((n_tokens,), jnp.int32),
        mesh=mesh,
        scratch_types=[
            pltpu.VMEM((vocab,), jnp.int32),
            [pltpu.VMEM((CH,), jnp.int32)] * 2,
            [pltpu.VMEM((CH,), jnp.int32)] * 2,
            [pltpu.SemaphoreType.DMA] * 2,
            [pltpu.SemaphoreType.DMA] * 2,
        ],
        compiler_params=pltpu.CompilerParams(needs_layout_passes=False),
    )
    def sc_kernel(ids_hbm, table_hbm, next_hbm, table_v, ids_v, next_v,
                  sem_in, sem_out):
        wid = lax.axis_index("s") * NC + lax.axis_index("c")
        base = wid * per_w
        pltpu.sync_copy(table_hbm, table_v)

        def start_in(c):
            return pltpu.async_copy(
                ids_hbm.at[pl.ds(base + c * CH, CH)], ids_v[c % 2], sem_in[c % 2])

        in_cp = [start_in(0), None]
        out_cp = [None, None]
        for c in range(4):
            if c < 3:
                in_cp[(c + 1) % 2] = start_in(c + 1)
            in_cp[c % 2].wait()
            if out_cp[c % 2] is not None:
                out_cp[c % 2].wait()

            @pl.loop(0, CH // L, unroll=8)
            def _(j, _c=c):
                ids16 = ids_v[_c % 2][pl.ds(j * L, L)]
                next_v[_c % 2][pl.ds(j * L, L)] = plsc.load_gather(table_v, [ids16])

            out_cp[c % 2] = pltpu.async_copy(
                next_v[c % 2], next_hbm.at[pl.ds(base + c * CH, CH)], sem_out[c % 2])
        out_cp[0].wait()
        out_cp[1].wait()

    return sc_kernel


def _tc_onehot(batch: int, seq: int, vocab: int):
    def body(fill_ref, nx_ref, out_ref):
        f0 = fill_ref[0]
        f1 = fill_ref[1]
        vio = lax.broadcasted_iota(jnp.int32, (vocab, 128), 0)
        for g in range(SB // 128):
            row = nx_ref[0, pl.ds(g, 1), :]
            bc = jnp.broadcast_to(row, (vocab, 128))
            out_ref[0, :, pl.ds(g * 128, 128)] = jnp.where(bc == vio, f1, f0)

    return pl.pallas_call(
        body,
        grid=(batch,),
        in_specs=[
            pl.BlockSpec(memory_space=pltpu.SMEM),
            pl.BlockSpec((1, SB // 128, 128), lambda b: (b, 0, 0)),
        ],
        out_specs=pl.BlockSpec((1, vocab, SB), lambda b: (b, 0, 0)),
        out_shape=jax.ShapeDtypeStruct((batch, vocab, seq), jnp.float32),
    )


def kernel(input_ids, transition_table, fill_values):
    batch, seq = input_ids.shape
    vocab = transition_table.shape[0]
    n = batch * seq
    ids_flat = input_ids.reshape(n)
    next_flat = _sc_lookup(n, vocab)(ids_flat, transition_table)
    nx3 = next_flat.reshape(batch, seq // 128, 128)
    out_t = _tc_onehot(batch, seq, vocab)(fill_values, nx3)
    return jnp.transpose(out_t, (0, 2, 1))


# speedup vs baseline: 5.5969x; 1.0135x over previous
import functools

import jax
import jax.numpy as jnp
from jax import lax
from jax.experimental import pallas as pl
from jax.experimental.pallas import tpu as pltpu
from jax.experimental.pallas import tpu_sc as plsc

L = 16
NC = 2
NS = 16
NW = NC * NS


def _sc_lookup(n_tokens: int, vocab: int, n_half: int, half: int):
    per_w = n_half // NW
    CH = per_w // 2
    mesh = plsc.VectorSubcoreMesh(core_axis_name="c", subcore_axis_name="s")

    @functools.partial(
        pl.kernel,
        out_type=jax.ShapeDtypeStruct((n_half,), jnp.int32),
        mesh=mesh,
        scratch_types=[
            pltpu.VMEM((vocab,), jnp.int32),
            [pltpu.VMEM((CH,), jnp.int32)] * 2,
            [pltpu.VMEM((CH,), jnp.int32)] * 2,
            [pltpu.SemaphoreType.DMA] * 2,
            [pltpu.SemaphoreType.DMA] * 2,
        ],
        compiler_params=pltpu.CompilerParams(needs_layout_passes=False),
    )
    def sc_kernel(ids_hbm, table_hbm, next_hbm, table_v, ids_v, next_v,
                  sem_in, sem_out):
        wid = lax.axis_index("s") * NC + lax.axis_index("c")
        in_base = half * n_half + wid * per_w
        out_base = wid * per_w
        pltpu.sync_copy(table_hbm, table_v)

        def start_in(c):
            return pltpu.async_copy(
                ids_hbm.at[pl.ds(in_base + c * CH, CH)], ids_v[c % 2],
                sem_in[c % 2])

        in_cp = [start_in(0), None]
        out_cp = [None, None]
        for c in range(2):
            if c < 1:
                in_cp[(c + 1) % 2] = start_in(c + 1)
            in_cp[c % 2].wait()

            @pl.loop(0, CH // L, unroll=8)
            def _(j, _c=c):
                ids16 = ids_v[_c % 2][pl.ds(j * L, L)]
                next_v[_c % 2][pl.ds(j * L, L)] = plsc.load_gather(
                    table_v, [ids16])

            out_cp[c % 2] = pltpu.async_copy(
                next_v[c % 2], next_hbm.at[pl.ds(out_base + c * CH, CH)],
                sem_out[c % 2])
        out_cp[0].wait()
        out_cp[1].wait()

    return sc_kernel


def _tc_onehot_half(batch: int, seq: int, vocab: int, half: int):
    hb = batch // 2
    row0 = half * hb

    def body(fill_ref, nx_ref, *refs):
        out_ref = refs[-1]
        f0 = fill_ref[0]
        f1 = fill_ref[1]
        vio = lax.broadcasted_iota(jnp.int32, (vocab, 128), 0)
        for g in range(seq // 128):
            row = nx_ref[0, pl.ds(g, 1), :]
            bc = jnp.broadcast_to(row, (vocab, 128))
            out_ref[0, :, pl.ds(g * 128, 128)] = jnp.where(bc == vio, f1, f0)

    in_specs = [
        pl.BlockSpec(memory_space=pltpu.SMEM),
        pl.BlockSpec((1, seq // 128, 128), lambda b: (b, 0, 0)),
    ]
    kwargs = {}
    if half:
        in_specs.append(pl.BlockSpec(memory_space=pltpu.MemorySpace.HBM))
        kwargs["input_output_aliases"] = {2: 0}
    return pl.pallas_call(
        body,
        grid=(hb,),
        in_specs=in_specs,
        out_specs=pl.BlockSpec((1, vocab, seq), lambda b: (b + row0, 0, 0)),
        out_shape=jax.ShapeDtypeStruct((batch, vocab, seq), jnp.float32),
        **kwargs,
    )


def kernel(input_ids, transition_table, fill_values):
    batch, seq = input_ids.shape
    vocab = transition_table.shape[0]
    n = batch * seq
    n_half = n // 2
    ids_flat = input_ids.reshape(n)
    next0 = _sc_lookup(n, vocab, n_half, 0)(ids_flat, transition_table)
    next1 = _sc_lookup(n, vocab, n_half, 1)(ids_flat, transition_table)
    nxa = next0.reshape(batch // 2, seq // 128, 128)
    nxb = next1.reshape(batch // 2, seq // 128, 128)
    out_t = _tc_onehot_half(batch, seq, vocab, 0)(fill_values, nxa)
    out_t = _tc_onehot_half(batch, seq, vocab, 1)(fill_values, nxb, out_t)
    return jnp.transpose(out_t, (0, 2, 1))
